# baseline (speedup 1.0000x reference)
import jax
import jax.numpy as jnp
from jax.experimental import pallas as pl
from jax.experimental.pallas import tpu as pltpu

EMOJI_VOCAB = 100
EMOJI_DIM = 50
SPACY_DIM = 300
OUT_DIM = SPACY_DIM + EMOJI_DIM

_BB = 64


def _block_kernel(spacy_ref, ids_ref, table_ref, out_ref):
    out_ref[:, :, :SPACY_DIM] = spacy_ref[...]
    bb, seq = ids_ref.shape
    table = table_ref[...]
    vocab_iota = jax.lax.broadcasted_iota(
        jnp.int32, (bb, seq, EMOJI_VOCAB), 2)
    onehot = (ids_ref[...][:, :, None] == vocab_iota).astype(jnp.float32)
    for i in range(bb):
        emoji = jax.lax.dot_general(
            onehot[i], table, (((1,), (0,)), ((), ())),
            preferred_element_type=jnp.float32)
        out_ref[i, :, SPACY_DIM:] = emoji


def kernel(spacy_vectors, emoji_ids, emoji_table):
    b, s, d = spacy_vectors.shape
    grid = (b // _BB,)
    return pl.pallas_call(
        _block_kernel,
        grid=grid,
        in_specs=[
            pl.BlockSpec((_BB, s, d), lambda i: (i, 0, 0)),
            pl.BlockSpec((_BB, s), lambda i: (i, 0)),
            pl.BlockSpec((EMOJI_VOCAB, EMOJI_DIM), lambda i: (0, 0)),
        ],
        out_specs=pl.BlockSpec((_BB, s, OUT_DIM), lambda i: (i, 0, 0)),
        out_shape=jax.ShapeDtypeStruct((b, s, OUT_DIM), jnp.float32),
    )(spacy_vectors, emoji_ids, emoji_table)
